# baseline (speedup 1.0000x reference)
import functools

import jax
import jax.numpy as jnp
from jax import lax
from jax.experimental import pallas as pl
from jax.experimental.pallas import tpu as pltpu
from jax.experimental.pallas import tpu_sc as plsc

_N = 10000
_D = 128
_NC = 2
_NS = 16
_NW = _NC * _NS
_B = 128
_NPAD = 10112
_ZR = _NPAD // _NS


def _deg_body(dst_hbm, zeros_hbm, ones_hbm, degp_hbm, dst_v, ones_v, acc,
              sem):
    c = lax.axis_index("c")
    s = lax.axis_index("s")
    w = c * _NS + s
    ch = dst_v.shape[0]
    pltpu.sync_copy(dst_hbm.at[w], dst_v)
    pltpu.sync_copy(ones_hbm, ones_v)
    pltpu.sync_copy(zeros_hbm, acc.at[pl.ds(s * _ZR, _ZR)])
    plsc.subcore_barrier()

    def step(j, carry):
        pltpu.async_copy(ones_v, acc.at[dst_v.at[j]], sem, add=True)
        return carry

    lax.fori_loop(0, ch, step, 0)

    def drain(j, carry):
        pltpu.make_async_copy(ones_v, acc.at[dst_v.at[j]], sem).wait()
        return carry

    lax.fori_loop(0, ch, drain, 0)
    plsc.subcore_barrier()
    pltpu.sync_copy(acc.at[pl.ds(s * _ZR, _ZR)],
                    degp_hbm.at[c, pl.ds(s * _ZR, _ZR)])


def _seg_body(hs_hbm, src_hbm, dst_hbm, zeros_hbm, segp_hbm,
              src_v, dst_v, rows0, acc, gsem0):
    c = lax.axis_index("c")
    s = lax.axis_index("s")
    w = c * _NS + s
    g_win = src_v.shape[0]

    pltpu.sync_copy(zeros_hbm, acc.at[pl.ds(s * _ZR, _ZR)])
    plsc.subcore_barrier()

    def step(j, carry):
        pltpu.async_copy(hs_hbm.at[src_v.at[j]], rows0, gsem0).wait()
        pltpu.sync_copy(rows0, acc.at[dst_v.at[j]], add=True)
        return carry

    def run_window(base):
        pltpu.sync_copy(src_hbm.at[w, pl.ds(base, g_win)], src_v)
        pltpu.sync_copy(dst_hbm.at[w, pl.ds(base, g_win)], dst_v)
        lax.fori_loop(0, g_win, step, 0)

    run_window(0)
    run_window(g_win)
    plsc.subcore_barrier()
    pltpu.sync_copy(acc.at[pl.ds(s * _ZR, _ZR)],
                    segp_hbm.at[c, pl.ds(s * _ZR, _ZR)])


def _deg_call(dst3, ch):
    mesh = plsc.VectorSubcoreMesh(core_axis_name="c", subcore_axis_name="s")
    zeros = jnp.zeros((_ZR, _D), jnp.float32)
    ones = jnp.ones((_B, _D), jnp.float32)
    return pl.kernel(
        _deg_body,
        out_type=jax.ShapeDtypeStruct((_NC, _NPAD, _D), jnp.float32),
        mesh=mesh,
        scratch_types=[
            pltpu.VMEM((ch, _B), jnp.int32),
            pltpu.VMEM((_B, _D), jnp.float32),
            pltpu.VMEM_SHARED((_NPAD, _D), jnp.float32),
            pltpu.SemaphoreType.DMA,
        ],
    )(dst3, zeros, ones)


def _seg_call(hs, src3, dst3, ch):
    mesh = plsc.VectorSubcoreMesh(core_axis_name="c", subcore_axis_name="s")
    zeros = jnp.zeros((_ZR, _D), jnp.float32)
    return pl.kernel(
        _seg_body,
        out_type=jax.ShapeDtypeStruct((_NC, _NPAD, _D), jnp.float32),
        mesh=mesh,
        scratch_types=[
            pltpu.VMEM((ch // 2, _B), jnp.int32),
            pltpu.VMEM((ch // 2, _B), jnp.int32),
            pltpu.VMEM((_B, _D), jnp.float32),
            pltpu.VMEM_SHARED((_NPAD, _D), jnp.float32),
            pltpu.SemaphoreType.DMA,
        ],
    )(hs, src3, dst3, zeros)


_RB = 1000


def _dinv_of(d0, d1):
    cnt = d0[:, 0:1] + d1[:, 0:1] + 1.0
    return lax.rsqrt(cnt)


def _tc1_body(x_ref, w_ref, d0_ref, d1_ref, hs_ref):
    h = jnp.dot(x_ref[...], w_ref[...], preferred_element_type=jnp.float32)
    hs_ref[...] = h * _dinv_of(d0_ref[...], d1_ref[...])


def _ln_relu(agg, g, b):
    mu = jnp.mean(agg, axis=1, keepdims=True)
    var = jnp.mean(jnp.square(agg - mu), axis=1, keepdims=True)
    y = (agg - mu) * lax.rsqrt(var + 1e-5) * g + b
    return jnp.maximum(y, 0.0)


def _tc2_body(s0_ref, s1_ref, hs_ref, d0_ref, d1_ref, b1_ref, g1_ref,
              bt1_ref, w2_ref, out_ref):
    dinv = _dinv_of(d0_ref[...], d1_ref[...])
    agg = (s0_ref[...] + s1_ref[...] + hs_ref[...]) * dinv + b1_ref[...]
    y = _ln_relu(agg, g1_ref[...], bt1_ref[...])
    h2 = jnp.dot(y, w2_ref[...], preferred_element_type=jnp.float32)
    out_ref[...] = h2 * dinv


def _tc3_body(s0_ref, s1_ref, hs_ref, d0_ref, d1_ref, b2_ref, g2_ref,
              bt2_ref, x_ref, out_ref):
    dinv = _dinv_of(d0_ref[...], d1_ref[...])
    agg = (s0_ref[...] + s1_ref[...] + hs_ref[...]) * dinv + b2_ref[...]
    y = _ln_relu(agg, g2_ref[...], bt2_ref[...])
    out_ref[...] = y + x_ref[...]


def _row_spec(width=_D):
    return pl.BlockSpec((_RB, width), lambda i: (i, 0))


def _full_spec(shape):
    return pl.BlockSpec(shape, lambda i: tuple(0 for _ in shape))


def _tc1(x, w1, d0, d1):
    return pl.pallas_call(
        _tc1_body,
        grid=(_N // _RB,),
        in_specs=[_row_spec(), _full_spec((_D, _D)), _row_spec(),
                  _row_spec()],
        out_specs=_row_spec(),
        out_shape=jax.ShapeDtypeStruct((_N, _D), jnp.float32),
    )(x, w1, d0, d1)


def _tc2(s0, s1, hs, d0, d1, b1, g1, bt1, w2):
    return pl.pallas_call(
        _tc2_body,
        grid=(_N // _RB,),
        in_specs=[_row_spec(), _row_spec(), _row_spec(), _row_spec(),
                  _row_spec(), _full_spec((1, _D)), _full_spec((1, _D)),
                  _full_spec((1, _D)), _full_spec((_D, _D))],
        out_specs=_row_spec(),
        out_shape=jax.ShapeDtypeStruct((_N, _D), jnp.float32),
    )(s0, s1, hs, d0, d1, b1, g1, bt1, w2)


def _tc3(s0, s1, hs, d0, d1, b2, g2, bt2, x):
    return pl.pallas_call(
        _tc3_body,
        grid=(_N // _RB,),
        in_specs=[_row_spec(), _row_spec(), _row_spec(), _row_spec(),
                  _row_spec(), _full_spec((1, _D)), _full_spec((1, _D)),
                  _full_spec((1, _D)), _row_spec()],
        out_specs=_row_spec(),
        out_shape=jax.ShapeDtypeStruct((_N, _D), jnp.float32),
    )(s0, s1, hs, d0, d1, b2, g2, bt2, x)


def kernel(x, edge_index, W1, b1, ln1_g, ln1_b, W2, b2, ln2_g, ln2_b):
    e = edge_index.shape[1]
    src = edge_index[0].astype(jnp.int32)
    dst = edge_index[1].astype(jnp.int32)
    ch = -(-e // (_NW * _B))
    ch = -(-ch // 4) * 4
    epad = _NW * ch * _B
    pad = epad - e
    src3 = jnp.concatenate(
        [src, jnp.zeros((pad,), jnp.int32)]).reshape(_NW, ch, _B)
    dst3 = jnp.concatenate(
        [dst, jnp.full((pad,), _N, jnp.int32)]).reshape(_NW, ch, _B)

    degp = _deg_call(dst3, ch)
    d0, d1 = degp[0], degp[1]

    b1r = b1.reshape(1, _D)
    b2r = b2.reshape(1, _D)
    g1r = ln1_g.reshape(1, _D)
    bt1r = ln1_b.reshape(1, _D)
    g2r = ln2_g.reshape(1, _D)
    bt2r = ln2_b.reshape(1, _D)

    hs1 = _tc1(x, W1, d0, d1)
    segp1 = _seg_call(hs1, src3, dst3, ch)
    hs2 = _tc2(segp1[0], segp1[1], hs1, d0, d1, b1r, g1r, bt1r, W2)
    segp2 = _seg_call(hs2, src3, dst3, ch)
    out = _tc3(segp2[0], segp2[1], hs2, d0, d1, b2r, g2r, bt2r, x)
    return out


# speedup vs baseline: 1.4693x; 1.4693x over previous
import functools

import jax
import jax.numpy as jnp
from jax import lax
from jax.experimental import pallas as pl
from jax.experimental.pallas import tpu as pltpu
from jax.experimental.pallas import tpu_sc as plsc

_N = 10000
_D = 128
_NC = 2
_NS = 16
_NW = _NC * _NS
_B = 128
_NPAD = 10240
_ZR = _NPAD // _NS


def _deg_body(dst_hbm, zeros_hbm, ones_hbm, degp_hbm, dst_v, ones_v, acc):
    c = lax.axis_index("c")
    s = lax.axis_index("s")
    w = c * _NS + s
    ch = dst_v.shape[0]
    pltpu.sync_copy(dst_hbm.at[w], dst_v)
    pltpu.sync_copy(ones_hbm, ones_v)
    pltpu.sync_copy(zeros_hbm, acc.at[pl.ds(s * _ZR, _ZR)])
    plsc.subcore_barrier()

    def step(j, carry):
        pltpu.sync_copy(ones_v, acc.at[dst_v.at[j]], add=True)
        return carry

    lax.fori_loop(0, ch, step, 0)
    plsc.subcore_barrier()
    pltpu.sync_copy(acc.at[pl.ds(s * _ZR, _ZR)],
                    degp_hbm.at[c, pl.ds(s * _ZR, _ZR)])


def _seg_body(hs_hbm, src_hbm, dst_hbm, zeros_hbm, segp_hbm,
              src_v, dst_v, rows0, acc, gsem0):
    c = lax.axis_index("c")
    s = lax.axis_index("s")
    w = c * _NS + s
    pltpu.sync_copy(zeros_hbm, acc.at[pl.ds(s * _ZR, _ZR)])
    plsc.subcore_barrier()

    pltpu.sync_copy(src_hbm.at[w], src_v)
    pltpu.sync_copy(dst_hbm.at[w], dst_v)

    def step(j, carry):
        pltpu.async_copy(hs_hbm.at[src_v.at[j]], rows0, gsem0).wait()
        pltpu.sync_copy(rows0, acc.at[dst_v.at[j]], add=True)
        return carry

    lax.fori_loop(0, src_v.shape[0], step, 0)
    plsc.subcore_barrier()
    pltpu.sync_copy(acc.at[pl.ds(s * _ZR, _ZR)],
                    segp_hbm.at[c, pl.ds(s * _ZR, _ZR)])


def _deg_call(dst3, ch):
    mesh = plsc.VectorSubcoreMesh(core_axis_name="c", subcore_axis_name="s")
    zeros = jnp.zeros((_ZR, _D), jnp.float32)
    ones = jnp.ones((_B, _D), jnp.float32)
    return pl.kernel(
        _deg_body,
        out_type=jax.ShapeDtypeStruct((_NC, _NPAD, _D), jnp.float32),
        mesh=mesh,
        scratch_types=[
            pltpu.VMEM((ch, _B), jnp.int32),
            pltpu.VMEM((_B, _D), jnp.float32),
            pltpu.VMEM_SHARED((_NPAD, _D), jnp.float32),
        ],
    )(dst3, zeros, ones)


def _seg_call(hs, src3, dst3, ch):
    mesh = plsc.VectorSubcoreMesh(core_axis_name="c", subcore_axis_name="s")
    zeros = jnp.zeros((_ZR, _D), jnp.float32)
    return pl.kernel(
        _seg_body,
        out_type=jax.ShapeDtypeStruct((_NC, _NPAD, _D), jnp.float32),
        mesh=mesh,
        scratch_types=[
            pltpu.VMEM((ch, _B), jnp.int32),
            pltpu.VMEM((ch, _B), jnp.int32),
            pltpu.VMEM((_B, _D), jnp.float32),
            pltpu.VMEM_SHARED((_NPAD, _D), jnp.float32),
            pltpu.SemaphoreType.DMA,
        ],
    )(hs, src3, dst3, zeros)


_RB = 1000


def _dinv_of(d0, d1):
    cnt = d0[:, 0:1] + d1[:, 0:1] + 1.0
    return lax.rsqrt(cnt)


def _tc1_body(x_ref, w_ref, d0_ref, d1_ref, hs_ref):
    h = jnp.dot(x_ref[...], w_ref[...], preferred_element_type=jnp.float32)
    hs_ref[...] = h * _dinv_of(d0_ref[...], d1_ref[...])


def _ln_relu(agg, g, b):
    mu = jnp.mean(agg, axis=1, keepdims=True)
    var = jnp.mean(jnp.square(agg - mu), axis=1, keepdims=True)
    y = (agg - mu) * lax.rsqrt(var + 1e-5) * g + b
    return jnp.maximum(y, 0.0)


def _tc2_body(s0_ref, s1_ref, hs_ref, d0_ref, d1_ref, b1_ref, g1_ref,
              bt1_ref, w2_ref, out_ref):
    dinv = _dinv_of(d0_ref[...], d1_ref[...])
    agg = (s0_ref[...] + s1_ref[...] + hs_ref[...]) * dinv + b1_ref[...]
    y = _ln_relu(agg, g1_ref[...], bt1_ref[...])
    h2 = jnp.dot(y, w2_ref[...], preferred_element_type=jnp.float32)
    out_ref[...] = h2 * dinv


def _tc3_body(s0_ref, s1_ref, hs_ref, d0_ref, d1_ref, b2_ref, g2_ref,
              bt2_ref, x_ref, out_ref):
    dinv = _dinv_of(d0_ref[...], d1_ref[...])
    agg = (s0_ref[...] + s1_ref[...] + hs_ref[...]) * dinv + b2_ref[...]
    y = _ln_relu(agg, g2_ref[...], bt2_ref[...])
    out_ref[...] = y + x_ref[...]


def _row_spec(width=_D):
    return pl.BlockSpec((_RB, width), lambda i: (i, 0))


def _full_spec(shape):
    return pl.BlockSpec(shape, lambda i: tuple(0 for _ in shape))


def _tc1(x, w1, d0, d1):
    return pl.pallas_call(
        _tc1_body,
        grid=(_N // _RB,),
        in_specs=[_row_spec(), _full_spec((_D, _D)), _row_spec(),
                  _row_spec()],
        out_specs=_row_spec(),
        out_shape=jax.ShapeDtypeStruct((_N, _D), jnp.float32),
    )(x, w1, d0, d1)


def _tc2(s0, s1, hs, d0, d1, b1, g1, bt1, w2):
    return pl.pallas_call(
        _tc2_body,
        grid=(_N // _RB,),
        in_specs=[_row_spec(), _row_spec(), _row_spec(), _row_spec(),
                  _row_spec(), _full_spec((1, _D)), _full_spec((1, _D)),
                  _full_spec((1, _D)), _full_spec((_D, _D))],
        out_specs=_row_spec(),
        out_shape=jax.ShapeDtypeStruct((_N, _D), jnp.float32),
    )(s0, s1, hs, d0, d1, b1, g1, bt1, w2)


def _tc3(s0, s1, hs, d0, d1, b2, g2, bt2, x):
    return pl.pallas_call(
        _tc3_body,
        grid=(_N // _RB,),
        in_specs=[_row_spec(), _row_spec(), _row_spec(), _row_spec(),
                  _row_spec(), _full_spec((1, _D)), _full_spec((1, _D)),
                  _full_spec((1, _D)), _row_spec()],
        out_specs=_row_spec(),
        out_shape=jax.ShapeDtypeStruct((_N, _D), jnp.float32),
    )(s0, s1, hs, d0, d1, b2, g2, bt2, x)


def kernel(x, edge_index, W1, b1, ln1_g, ln1_b, W2, b2, ln2_g, ln2_b):
    e = edge_index.shape[1]
    src = edge_index[0].astype(jnp.int32)
    dst = edge_index[1].astype(jnp.int32)
    ch = -(-e // (_NW * _B))
    epad = _NW * ch * _B
    pad = epad - e
    src3 = jnp.concatenate(
        [src, jnp.zeros((pad,), jnp.int32)]).reshape(_NW, ch, _B)
    dst3 = jnp.concatenate(
        [dst, jnp.full((pad,), _N, jnp.int32)]).reshape(_NW, ch, _B)

    degp = _deg_call(dst3, ch)
    d0, d1 = degp[0], degp[1]

    b1r = b1.reshape(1, _D)
    b2r = b2.reshape(1, _D)
    g1r = ln1_g.reshape(1, _D)
    bt1r = ln1_b.reshape(1, _D)
    g2r = ln2_g.reshape(1, _D)
    bt2r = ln2_b.reshape(1, _D)

    hs1 = _tc1(x, W1, d0, d1)
    segp1 = _seg_call(hs1, src3, dst3, ch)
    hs2 = _tc2(segp1[0], segp1[1], hs1, d0, d1, b1r, g1r, bt1r, W2)
    segp2 = _seg_call(hs2, src3, dst3, ch)
    out = _tc3(segp2[0], segp2[1], hs2, d0, d1, b2r, g2r, bt2r, x)
    return out
